# Initial kernel scaffold; baseline (speedup 1.0000x reference)
import jax
import jax.numpy as jnp
from jax.experimental import pallas as pl


def kernel(xyz, params):
    raise NotImplementedError("write your pallas kernel here")


# speedup vs baseline: 13.0871x; 13.0871x over previous
import functools

import jax
import jax.numpy as jnp
from jax.experimental import pallas as pl

_B = 8
_EPS = 1e-5


def _fold_bn(p):
    s = p['gamma'] / jnp.sqrt(p['var'] + _EPS)
    w = p['w'] * s[:, None]
    b = (p['b'] - p['mean']) * s + p['beta']
    return w, b.reshape(-1, 1)


def _incl_cumsum(x, n):
    sh = 1
    while sh < n:
        x = x + jnp.pad(x, ((0, 0), (sh, 0)))[:, :-sh]
        sh *= 2
    return x


def _searchsorted(cum, t, n):
    nch = n // 128
    r_dim = cum.shape[0]
    cumr = cum.reshape(r_dim, nch, 128)
    pos = jnp.zeros(t.shape, jnp.int32)
    b = n // 2
    while b >= 1:
        npos = pos + b
        j = npos - 1
        jc = jnp.minimum(j, n - 1)
        c = jc // 128
        l = jc % 128
        v = jnp.zeros(t.shape, cum.dtype)
        for i in range(nch):
            g = jnp.take_along_axis(cumr[:, i, :], l, axis=1)
            v = jnp.where(c == i, g, v)
        ok = (j <= n - 1) & (v < t)
        pos = jnp.where(ok, npos, pos)
        b //= 2
    return pos


def _flatten_rows(x):
    r = x.shape[0]
    return jnp.concatenate([x[i:i + 1, :] for i in range(r)], axis=1)


def _gather_cols(src, idx):
    c_dim, n = src.shape
    nch = n // 128
    srcr = src.reshape(c_dim, nch, 128)
    m = idx.shape[1]
    idxb = jnp.broadcast_to(idx, (c_dim, m))
    ch = idxb // 128
    l = idxb % 128
    out = jnp.zeros((c_dim, m), src.dtype)
    for i in range(nch):
        g = jnp.take_along_axis(srcr[:, i, :], l, axis=1)
        out = jnp.where(ch == i, g, out)
    return out


def _ball_query(d2, r2, k, n):
    maskf = (d2 <= r2).astype(jnp.float32)
    cum = _incl_cumsum(maskf, n)
    cnt = cum[:, n - 1:n]
    tk = jax.lax.broadcasted_iota(
        jnp.int32, (d2.shape[0], k), 1).astype(jnp.float32) + 1.0
    pos = _searchsorted(cum, tk, n)
    gi = jnp.where(tk <= cnt, pos, pos[:, :1])
    return gi


def _fps_body(xyz_ref, newxyz_ref, *, npoint, n):
    x = xyz_ref[:, 0, :]
    y = xyz_ref[:, 1, :]
    z = xyz_ref[:, 2, :]
    iota = jax.lax.broadcasted_iota(jnp.int32, (_B, n), 1)

    def body(i, carry):
        distance, farthest = carry
        oh = (iota == farthest).astype(jnp.float32)
        cx = jnp.sum(x * oh, axis=1, keepdims=True)
        cy = jnp.sum(y * oh, axis=1, keepdims=True)
        cz = jnp.sum(z * oh, axis=1, keepdims=True)
        newxyz_ref[:, pl.ds(i, 1), :] = jnp.concatenate(
            [cx, cy, cz], axis=1)[:, None, :]
        dx = x - cx
        dy = y - cy
        dz = z - cz
        dist = dx * dx + dy * dy + dz * dz
        distance = jnp.minimum(distance, dist)
        m = jnp.max(distance, axis=1, keepdims=True)
        cand = jnp.where(distance == m, iota, n)
        farthest = jnp.min(cand, axis=1, keepdims=True)
        return distance, farthest

    init = (jnp.full((_B, n), 1e10, jnp.float32),
            jnp.zeros((_B, 1), jnp.int32))
    jax.lax.fori_loop(0, npoint, body, init)


def _fps(xyz_t, npoint):
    n = xyz_t.shape[2]
    newxyz = pl.pallas_call(
        functools.partial(_fps_body, npoint=npoint, n=n),
        out_shape=jax.ShapeDtypeStruct((_B, npoint, 3), jnp.float32),
    )(xyz_t)
    return newxyz


_SA1_SCALES = ((0.1, 32), (0.2, 64), (0.4, 128))
_SA1_TILE = 128


def _sa1_body(xyz_ref, centT_ref, cent_ref, *refs):
    w = refs[:18]
    outs = refs[18:]
    pts = xyz_ref[0]
    ptsq = jnp.sum(pts * pts, axis=0, keepdims=True)
    c_t = centT_ref[0]
    c_r = cent_ref[0]
    censq = jnp.sum(c_r * c_r, axis=1, keepdims=True)
    cross = jnp.dot(c_r, pts, preferred_element_type=jnp.float32)
    d2 = censq - 2.0 * cross + ptsq

    for si, (radius, k) in enumerate(_SA1_SCALES):
        gi = _ball_query(d2, radius * radius, k, 4096)
        gif = _flatten_rows(gi)
        g = _gather_cols(pts, gif)
        crep = jnp.broadcast_to(
            c_t[:, :, None], (3, _SA1_TILE, k)).reshape(3, _SA1_TILE * k)
        h = jnp.concatenate([g, g - crep], axis=0)
        for li in range(3):
            wi = w[si * 6 + 2 * li][...]
            bi = w[si * 6 + 2 * li + 1][...]
            h = jnp.maximum(
                jnp.dot(wi, h, preferred_element_type=jnp.float32) + bi, 0.0)
        cout = h.shape[0]
        hm = jnp.max(h.reshape(cout, _SA1_TILE, k), axis=2)
        outs[si][0] = hm


def _sa1(xyz_t, newxyz_t, newxyz, wlist):
    n_tiles = 512 // _SA1_TILE
    full = lambda shape: pl.BlockSpec(shape, lambda b, t: (0,) * len(shape))
    in_specs = [
        pl.BlockSpec((1, 3, 4096), lambda b, t: (b, 0, 0)),
        pl.BlockSpec((1, 3, _SA1_TILE), lambda b, t: (b, 0, t)),
        pl.BlockSpec((1, _SA1_TILE, 3), lambda b, t: (b, t, 0)),
    ] + [full(wi.shape) for wi in wlist]
    out_specs = [
        pl.BlockSpec((1, c, _SA1_TILE), lambda b, t: (b, 0, t))
        for c in (64, 128, 128)
    ]
    out_shape = [jax.ShapeDtypeStruct((_B, c, 512), jnp.float32)
                 for c in (64, 128, 128)]
    return pl.pallas_call(
        _sa1_body,
        grid=(_B, n_tiles),
        in_specs=in_specs,
        out_specs=out_specs,
        out_shape=out_shape,
    )(xyz_t, newxyz_t, newxyz, *wlist)


_SA2_SCALES = ((0.4, 64), (0.8, 128))
_SA2_TILE = 128


def _sa2_body(xyz_ref, feats_ref, centT_ref, cent_ref, *refs):
    w = refs[:14]
    outs = refs[14:]
    pts = xyz_ref[0]
    feats = feats_ref[0]
    ptsq = jnp.sum(pts * pts, axis=0, keepdims=True)
    c_t = centT_ref[0]
    c_r = cent_ref[0]
    censq = jnp.sum(c_r * c_r, axis=1, keepdims=True)
    cross = jnp.dot(c_r, pts, preferred_element_type=jnp.float32)
    d2 = censq - 2.0 * cross + ptsq

    for si, (radius, k) in enumerate(_SA2_SCALES):
        w1p = w[si * 7][...]
        w1x = w[si * 7 + 1][...]
        b1 = w[si * 7 + 2][...]
        gi = _ball_query(d2, radius * radius, k, 512)
        gif = _flatten_rows(gi)
        a = (jnp.dot(w1p, feats, preferred_element_type=jnp.float32)
             + jnp.dot(w1x, pts, preferred_element_type=jnp.float32))
        bc = b1 - jnp.dot(w1x, c_t, preferred_element_type=jnp.float32)
        ag = _gather_cols(a, gif)
        brep = jnp.broadcast_to(
            bc[:, :, None],
            (bc.shape[0], _SA2_TILE, k)).reshape(bc.shape[0], _SA2_TILE * k)
        h = jnp.maximum(ag + brep, 0.0)
        for li in range(2):
            wi = w[si * 7 + 3 + 2 * li][...]
            bi = w[si * 7 + 4 + 2 * li][...]
            h = jnp.maximum(
                jnp.dot(wi, h, preferred_element_type=jnp.float32) + bi, 0.0)
        hm = jnp.max(h.reshape(256, _SA2_TILE, k), axis=2)
        outs[si][0] = hm


def _sa2(l1xyz_t, feats, newxyz_t, newxyz, wlist):
    full = lambda shape: pl.BlockSpec(shape, lambda b, t: (0,) * len(shape))
    in_specs = [
        pl.BlockSpec((1, 3, 512), lambda b, t: (b, 0, 0)),
        pl.BlockSpec((1, 320, 512), lambda b, t: (b, 0, 0)),
        pl.BlockSpec((1, 3, _SA2_TILE), lambda b, t: (b, 0, t)),
        pl.BlockSpec((1, _SA2_TILE, 3), lambda b, t: (b, t, 0)),
    ] + [full(wi.shape) for wi in wlist]
    out_specs = [
        pl.BlockSpec((1, 256, _SA2_TILE), lambda b, t: (b, 0, t))
        for _ in range(2)
    ]
    out_shape = [jax.ShapeDtypeStruct((_B, 256, 128), jnp.float32)
                 for _ in range(2)]
    return pl.pallas_call(
        _sa2_body,
        grid=(_B, 128 // _SA2_TILE),
        in_specs=in_specs,
        out_specs=out_specs,
        out_shape=out_shape,
    )(l1xyz_t, feats, newxyz_t, newxyz, *wlist)


def _sa3_body(pts_ref, xyz_ref, *refs):
    (w1, b1, w2, b2, w3, b3, f1w, f1b, f2w, f2b, out_ref) = refs
    cols = []
    for b in range(_B):
        cols.append(jnp.concatenate([xyz_ref[b], pts_ref[b]], axis=0))
    h = jnp.concatenate(cols, axis=1)
    for wi, bi in ((w1, b1), (w2, b2), (w3, b3)):
        h = jnp.maximum(
            jnp.dot(wi[...], h, preferred_element_type=jnp.float32)
            + bi[...], 0.0)
    hm = jnp.max(h.reshape(1024, _B, 128), axis=2)
    y = jnp.dot(f1w[...], hm, preferred_element_type=jnp.float32) + f1b[...]
    y = jnp.where(y >= 0, y, 0.01 * y)
    out_ref[...] = (jnp.dot(f2w[...], y, preferred_element_type=jnp.float32)
                    + f2b[...])


def _sa3_head(feats, l2xyz_t, wlist):
    return pl.pallas_call(
        _sa3_body,
        out_shape=jax.ShapeDtypeStruct((40, _B), jnp.float32),
    )(feats, l2xyz_t, *wlist)


def kernel(xyz, params):
    sa1_w = []
    for convs in params['sa1']:
        for p in convs:
            w, b = _fold_bn(p)
            sa1_w += [w, b]
    sa2_w = []
    for convs in params['sa2']:
        for p in convs:
            w, b = _fold_bn(p)
            if p['w'].shape[1] == 323:
                sa2_w += [w[:, :320], w[:, 320:], b]
            else:
                sa2_w += [w, b]
    sa3_w = []
    for p in params['sa3']:
        w, b = _fold_bn(p)
        sa3_w += [w, b]
    fc = params['fc']
    sa3_w += [fc['fc1_w'], fc['fc1_b'].reshape(-1, 1),
              fc['fc2_w'], fc['fc2_b'].reshape(-1, 1)]

    newxyz1 = _fps(xyz, 512)
    newxyz1_t = jnp.transpose(newxyz1, (0, 2, 1))
    l1 = _sa1(xyz, newxyz1_t, newxyz1, sa1_w)
    feats1 = jnp.concatenate(l1, axis=1)

    newxyz2 = _fps(newxyz1_t, 128)
    newxyz2_t = jnp.transpose(newxyz2, (0, 2, 1))
    l2 = _sa2(newxyz1_t, feats1, newxyz2_t, newxyz2, sa2_w)
    feats2 = jnp.concatenate(l2, axis=1)

    logits_t = _sa3_head(feats2, newxyz2_t, sa3_w)
    return jnp.transpose(logits_t)
